# baseline (speedup 1.0000x reference)
import functools

import jax
import jax.numpy as jnp
from jax import lax
from jax.experimental import pallas as pl
from jax.experimental.pallas import tpu as pltpu
from jax.experimental.pallas import tpu_sc as plsc

POOL_N = 64
TOPK_N = 8
LEN_N = 5
D_N = 768
H_N = 4096
B_N = 128
S_N = 197
T_OUT = 1 + TOPK_N * LEN_N + S_N

NT = 8
TH = H_N // NT

NW = 32
NB_SC = B_N // NW


def _dense_body(cls_ref, pk_ref, eps_ref,
                w1_ref, b1_ref, wm_ref, bm_ref, wv_ref, bv_ref,
                d1_ref, db1_ref, d2_ref, db2_ref,
                sim_ref, mean_ref, lv_ref, syn_ref, idx_ref, rs_ref,
                acc_mean, acc_lv, z_ref, acc_syn):
    i = pl.program_id(0)

    @pl.when(i == 0)
    def _init():
        acc_mean[...] = jnp.zeros_like(acc_mean)
        acc_lv[...] = jnp.zeros_like(acc_lv)
        acc_syn[...] = jnp.zeros_like(acc_syn)
        x = cls_ref[...]
        xn = x * lax.rsqrt(jnp.maximum(
            jnp.sum(x * x, axis=1, keepdims=True), 1e-12))
        p = pk_ref[...]
        pn = p * lax.rsqrt(jnp.maximum(
            jnp.sum(p * p, axis=1, keepdims=True), 1e-12))
        sim = lax.dot_general(xn, pn, (((1,), (1,)), ((), ())),
                              preferred_element_type=jnp.float32)
        sim_ref[...] = sim
        iot = lax.broadcasted_iota(jnp.int32, (B_N, POOL_N), 1)
        work = sim
        total = jnp.float32(0.0)
        cols = []
        for _ in range(TOPK_N):
            m = jnp.max(work, axis=1, keepdims=True)
            total = total + jnp.sum(m)
            cand = jnp.where(work == m, iot, POOL_N)
            aidx = jnp.min(cand, axis=1)
            cols.append(aidx.reshape(B_N, 1))
            work = jnp.where(iot == aidx[:, None], -jnp.inf, work)
        idx_ref[...] = jnp.concatenate(cols, axis=1)
        rs_ref[...] = jnp.reshape(total / jnp.float32(B_N), (1, 1))

    @pl.when(i < NT)
    def _encoder():
        h = jnp.maximum(
            lax.dot_general(cls_ref[...], w1_ref[...],
                            (((1,), (1,)), ((), ())),
                            preferred_element_type=jnp.float32)
            + b1_ref[...][None, :], 0.0)
        acc_mean[...] += lax.dot_general(h, wm_ref[...],
                                         (((1,), (1,)), ((), ())),
                                         preferred_element_type=jnp.float32)
        acc_lv[...] += lax.dot_general(h, wv_ref[...],
                                       (((1,), (1,)), ((), ())),
                                       preferred_element_type=jnp.float32)

    @pl.when(i == NT - 1)
    def _reparam():
        mean = acc_mean[...] + bm_ref[...][None, :]
        log_var = acc_lv[...] + bv_ref[...][None, :]
        mean_ref[...] = mean
        lv_ref[...] = log_var
        z_ref[...] = mean + jnp.exp(0.5 * log_var) * eps_ref[...]

    @pl.when(i >= NT)
    def _decoder():
        hd = jnp.maximum(
            lax.dot_general(z_ref[...], d1_ref[...],
                            (((1,), (1,)), ((), ())),
                            preferred_element_type=jnp.float32)
            + db1_ref[...][None, :], 0.0)
        acc_syn[...] += lax.dot_general(hd, d2_ref[...],
                                        (((1,), (1,)), ((), ())),
                                        preferred_element_type=jnp.float32)

    @pl.when(i == 2 * NT - 1)
    def _finish():
        syn_ref[...] = acc_syn[...] + db2_ref[...][None, :]


_SC_UNITS = [
    (40, 72, 0, 72, 0),
    (112, 72, 64, 80, 7),
    (184, 54, 136, 61, 7),
]


def _sc_copy_body(x_ref, cls_ref, out_ref, buf, sem_in, sem_out):
    wid = lax.axis_index("s") * 2 + lax.axis_index("c")

    units = []
    for j in range(NB_SC):
        for spec in _SC_UNITS:
            units.append((j,) + spec)
    nu = len(units)

    def _al(off):
        return pl.multiple_of(off, D_N)

    def start_in(u):
        j, doff, drows, soff, srows, boff = units[u]
        b = wid * NB_SC + j
        sl = sem_in.at[u % 2]
        copies = []
        if doff == 40:
            copies.append(pltpu.async_copy(
                cls_ref.at[pl.ds(_al(b * D_N), D_N)],
                buf.at[u % 2, pl.ds(0, D_N)], sl))
            copies.append(pltpu.async_copy(
                x_ref.at[pl.ds(_al(b * S_N * D_N), srows * D_N)],
                buf.at[u % 2, pl.ds(D_N, srows * D_N)], sl))
        else:
            copies.append(pltpu.async_copy(
                x_ref.at[pl.ds(_al(b * S_N * D_N + soff * D_N),
                               srows * D_N)],
                buf.at[u % 2, pl.ds(0, srows * D_N)], sl))
        return copies

    def start_out(u):
        j, doff, drows, soff, srows, boff = units[u]
        b = wid * NB_SC + j
        return pltpu.async_copy(
            buf.at[u % 2, pl.ds(boff * D_N, drows * D_N)],
            out_ref.at[pl.ds(_al(b * T_OUT * D_N + doff * D_N),
                             drows * D_N)],
            sem_out.at[u % 2])

    hin = {0: start_in(0)}
    hout = {}
    for u in range(nu):
        for h in hin[u]:
            h.wait()
        hout[u] = start_out(u)
        if u + 1 < nu:
            if u >= 1:
                hout[u - 1].wait()
            hin[u + 1] = start_in(u + 1)
    hout[nu - 2].wait()
    hout[nu - 1].wait()


def _head_body(idx_sref, prompt_ref, syn_ref, outin_ref, out_ref, head, sem):
    del outin_ref

    def bloop(b, carry):
        s = syn_ref[b]
        for k in range(TOPK_N):
            pidx = idx_sref[b, k]
            head[b, k * LEN_N:(k + 1) * LEN_N, :] = prompt_ref[pidx] + s
        return carry

    lax.fori_loop(0, B_N, bloop, 0)
    cp = pltpu.make_async_copy(
        head, out_ref.at[:, pl.ds(0, TOPK_N * LEN_N), :], sem)
    cp.start()
    cp.wait()


def kernel(is_training, x_embed, cls_features, prompt, prompt_key, frequency,
           W1, b1, Wm, bm, Wv, bv, D1, db1, D2, db2, epsilon):
    del is_training, frequency

    enc_t = lambda i: jnp.where(i < NT, i, 0)
    dec_t = lambda i: jnp.where(i >= NT, i - NT, 0)

    sim, mean, log_var, synth, idx, rs = pl.pallas_call(
        _dense_body,
        grid=(2 * NT,),
        in_specs=[
            pl.BlockSpec((B_N, D_N), lambda i: (0, 0)),
            pl.BlockSpec((POOL_N, D_N), lambda i: (0, 0)),
            pl.BlockSpec((B_N, D_N), lambda i: (0, 0)),
            pl.BlockSpec((TH, D_N), lambda i: (enc_t(i), 0)),
            pl.BlockSpec((TH,), lambda i: (enc_t(i),)),
            pl.BlockSpec((D_N, TH), lambda i: (0, enc_t(i))),
            pl.BlockSpec((D_N,), lambda i: (0,)),
            pl.BlockSpec((D_N, TH), lambda i: (0, enc_t(i))),
            pl.BlockSpec((D_N,), lambda i: (0,)),
            pl.BlockSpec((TH, D_N), lambda i: (dec_t(i), 0)),
            pl.BlockSpec((TH,), lambda i: (dec_t(i),)),
            pl.BlockSpec((D_N, TH), lambda i: (0, dec_t(i))),
            pl.BlockSpec((D_N,), lambda i: (0,)),
        ],
        out_specs=[
            pl.BlockSpec((B_N, POOL_N), lambda i: (0, 0)),
            pl.BlockSpec((B_N, D_N), lambda i: (0, 0)),
            pl.BlockSpec((B_N, D_N), lambda i: (0, 0)),
            pl.BlockSpec((B_N, D_N), lambda i: (0, 0)),
            pl.BlockSpec((B_N, TOPK_N), lambda i: (0, 0)),
            pl.BlockSpec((1, 1), lambda i: (0, 0)),
        ],
        out_shape=[
            jax.ShapeDtypeStruct((B_N, POOL_N), jnp.float32),
            jax.ShapeDtypeStruct((B_N, D_N), jnp.float32),
            jax.ShapeDtypeStruct((B_N, D_N), jnp.float32),
            jax.ShapeDtypeStruct((B_N, D_N), jnp.float32),
            jax.ShapeDtypeStruct((B_N, TOPK_N), jnp.int32),
            jax.ShapeDtypeStruct((1, 1), jnp.float32),
        ],
        scratch_shapes=[
            pltpu.VMEM((B_N, D_N), jnp.float32),
            pltpu.VMEM((B_N, D_N), jnp.float32),
            pltpu.VMEM((B_N, D_N), jnp.float32),
            pltpu.VMEM((B_N, D_N), jnp.float32),
        ],
    )(cls_features, prompt_key, epsilon,
      W1, b1, Wm, bm, Wv, bv, D1, db1, D2, db2)

    sc_copy = functools.partial(
        pl.kernel,
        out_type=jax.ShapeDtypeStruct((B_N * T_OUT * D_N,), jnp.float32),
        mesh=plsc.VectorSubcoreMesh(core_axis_name="c", subcore_axis_name="s"),
        scratch_types=[
            pltpu.VMEM((2, 81 * D_N), jnp.float32),
            pltpu.SemaphoreType.DMA((2,)),
            pltpu.SemaphoreType.DMA((2,)),
        ],
    )(_sc_copy_body)
    out0 = sc_copy(x_embed.reshape(-1), cls_features.reshape(-1))
    out0 = out0.reshape(B_N, T_OUT, D_N)

    prompted = pl.pallas_call(
        _head_body,
        grid_spec=pltpu.PrefetchScalarGridSpec(
            num_scalar_prefetch=1,
            grid=(1,),
            in_specs=[
                pl.BlockSpec((POOL_N, LEN_N, D_N), lambda g, idx: (0, 0, 0)),
                pl.BlockSpec((B_N, 1, D_N), lambda g, idx: (0, 0, 0)),
                pl.BlockSpec(memory_space=pl.ANY),
            ],
            out_specs=pl.BlockSpec(memory_space=pl.ANY),
            scratch_shapes=[
                pltpu.VMEM((B_N, TOPK_N * LEN_N, D_N), jnp.float32),
                pltpu.SemaphoreType.DMA,
            ],
        ),
        out_shape=jax.ShapeDtypeStruct((B_N, T_OUT, D_N), jnp.float32),
        input_output_aliases={3: 0},
    )(idx, prompt, synth.reshape(B_N, 1, D_N), out0)

    return (prompted, rs.reshape(()), sim, synth, mean, log_var, idx)


# speedup vs baseline: 1.5967x; 1.5967x over previous
import functools

import jax
import jax.numpy as jnp
from jax import lax
from jax.experimental import pallas as pl
from jax.experimental.pallas import tpu as pltpu
from jax.experimental.pallas import tpu_sc as plsc

POOL_N = 64
TOPK_N = 8
LEN_N = 5
D_N = 768
H_N = 4096
B_N = 128
S_N = 197
T_OUT = 1 + TOPK_N * LEN_N + S_N

NT = 8
TH = H_N // NT

NW = 32
NB_SC = B_N // NW


def _dense_body(cls_ref, pk_ref, eps_ref,
                w1_ref, b1_ref, wm_ref, bm_ref, wv_ref, bv_ref,
                d1_ref, db1_ref, d2_ref, db2_ref,
                sim_ref, mean_ref, lv_ref, syn_ref, idx_ref, rs_ref,
                acc_mean, acc_lv, z_ref, acc_syn):
    i = pl.program_id(0)

    @pl.when(i == 0)
    def _init():
        acc_mean[...] = jnp.zeros_like(acc_mean)
        acc_lv[...] = jnp.zeros_like(acc_lv)
        acc_syn[...] = jnp.zeros_like(acc_syn)
        x = cls_ref[...]
        xn = x * lax.rsqrt(jnp.maximum(
            jnp.sum(x * x, axis=1, keepdims=True), 1e-12))
        p = pk_ref[...]
        pn = p * lax.rsqrt(jnp.maximum(
            jnp.sum(p * p, axis=1, keepdims=True), 1e-12))
        sim = lax.dot_general(xn, pn, (((1,), (1,)), ((), ())),
                              preferred_element_type=jnp.float32)
        sim_ref[...] = sim
        iot = lax.broadcasted_iota(jnp.int32, (B_N, POOL_N), 1)
        work = sim
        total = jnp.float32(0.0)
        cols = []
        for _ in range(TOPK_N):
            m = jnp.max(work, axis=1, keepdims=True)
            total = total + jnp.sum(m)
            cand = jnp.where(work == m, iot, POOL_N)
            aidx = jnp.min(cand, axis=1)
            cols.append(aidx.reshape(B_N, 1))
            work = jnp.where(iot == aidx[:, None], -jnp.inf, work)
        idx_ref[...] = jnp.concatenate(cols, axis=1)
        rs_ref[...] = jnp.reshape(total / jnp.float32(B_N), (1, 1))

    @pl.when(i < NT)
    def _encoder():
        h = jnp.maximum(
            lax.dot_general(cls_ref[...], w1_ref[...],
                            (((1,), (1,)), ((), ())),
                            preferred_element_type=jnp.float32)
            + b1_ref[...][None, :], 0.0)
        acc_mean[...] += lax.dot_general(h, wm_ref[...],
                                         (((1,), (1,)), ((), ())),
                                         preferred_element_type=jnp.float32)
        acc_lv[...] += lax.dot_general(h, wv_ref[...],
                                       (((1,), (1,)), ((), ())),
                                       preferred_element_type=jnp.float32)

    @pl.when(i == NT - 1)
    def _reparam():
        mean = acc_mean[...] + bm_ref[...][None, :]
        log_var = acc_lv[...] + bv_ref[...][None, :]
        mean_ref[...] = mean
        lv_ref[...] = log_var
        z_ref[...] = mean + jnp.exp(0.5 * log_var) * eps_ref[...]

    @pl.when(i >= NT)
    def _decoder():
        hd = jnp.maximum(
            lax.dot_general(z_ref[...], d1_ref[...],
                            (((1,), (1,)), ((), ())),
                            preferred_element_type=jnp.float32)
            + db1_ref[...][None, :], 0.0)
        acc_syn[...] += lax.dot_general(hd, d2_ref[...],
                                        (((1,), (1,)), ((), ())),
                                        preferred_element_type=jnp.float32)

    @pl.when(i == 2 * NT - 1)
    def _finish():
        syn_ref[...] = acc_syn[...] + db2_ref[...][None, :]


_SC_UNITS = [
    (40, 72, 0, 72, 0),
    (112, 72, 64, 80, 7),
    (184, 54, 136, 61, 7),
]


def _sc_copy_body(x_ref, cls_ref, out_ref, buf, sem_in, sem_out):
    wid = lax.axis_index("s") * 2 + lax.axis_index("c")

    units = []
    for j in range(NB_SC):
        for spec in _SC_UNITS:
            units.append((j,) + spec)
    nu = len(units)

    def _al(off):
        return pl.multiple_of(off, D_N)

    def start_in(u):
        j, doff, drows, soff, srows, boff = units[u]
        b = wid * NB_SC + j
        sl = sem_in.at[u % 2]
        copies = []
        if doff == 40:
            copies.append(pltpu.async_copy(
                cls_ref.at[pl.ds(_al(b * D_N), D_N)],
                buf.at[u % 2, pl.ds(0, D_N)], sl))
            copies.append(pltpu.async_copy(
                x_ref.at[pl.ds(_al(b * S_N * D_N), srows * D_N)],
                buf.at[u % 2, pl.ds(D_N, srows * D_N)], sl))
        else:
            copies.append(pltpu.async_copy(
                x_ref.at[pl.ds(_al(b * S_N * D_N + soff * D_N),
                               srows * D_N)],
                buf.at[u % 2, pl.ds(0, srows * D_N)], sl))
        return copies

    def start_out(u):
        j, doff, drows, soff, srows, boff = units[u]
        b = wid * NB_SC + j
        return pltpu.async_copy(
            buf.at[u % 2, pl.ds(boff * D_N, drows * D_N)],
            out_ref.at[pl.ds(_al(b * T_OUT * D_N + doff * D_N),
                             drows * D_N)],
            sem_out.at[u % 2])

    hin = {0: start_in(0)}
    hout = {}
    for u in range(nu):
        for h in hin[u]:
            h.wait()
        hout[u] = start_out(u)
        if u + 1 < nu:
            if u >= 1:
                hout[u - 1].wait()
            hin[u + 1] = start_in(u + 1)
    hout[nu - 2].wait()
    hout[nu - 1].wait()


BB = 8
NG = B_N // BB
NBUF = 4


def _assemble_body(idx_sref, prompt_ref, x_ref, syn_ref, cls_ref, out_ref,
                   obuf, sem):
    g = pl.program_id(0)
    buf = lax.rem(g, NBUF)

    @pl.when(g >= NBUF)
    def _drain():
        pltpu.make_async_copy(
            obuf.at[buf], out_ref.at[pl.ds((g - NBUF) * BB, BB)],
            sem.at[buf]).wait()

    obuf[buf, :, 41:, :] = x_ref[...]
    obuf[buf, :, 40:41, :] = cls_ref[...]
    for bb in range(BB):
        s = syn_ref[bb, 0, :]
        for k in range(TOPK_N):
            pidx = idx_sref[g * BB + bb, k]
            obuf[buf, bb, k * LEN_N:(k + 1) * LEN_N, :] = (
                prompt_ref[pidx] + s[None, :])

    pltpu.make_async_copy(
        obuf.at[buf], out_ref.at[pl.ds(g * BB, BB)], sem.at[buf]).start()

    @pl.when(g == NG - 1)
    def _final_drain():
        for d in range(NBUF - 1, -1, -1):
            gg = NG - 1 - d
            pltpu.make_async_copy(
                obuf.at[lax.rem(jnp.int32(gg), NBUF)],
                out_ref.at[pl.ds(gg * BB, BB)],
                sem.at[lax.rem(jnp.int32(gg), NBUF)]).wait()


def kernel(is_training, x_embed, cls_features, prompt, prompt_key, frequency,
           W1, b1, Wm, bm, Wv, bv, D1, db1, D2, db2, epsilon):
    del is_training, frequency

    enc_t = lambda i: jnp.where(i < NT, i, 0)
    dec_t = lambda i: jnp.where(i >= NT, i - NT, 0)

    sim, mean, log_var, synth, idx, rs = pl.pallas_call(
        _dense_body,
        grid=(2 * NT,),
        in_specs=[
            pl.BlockSpec((B_N, D_N), lambda i: (0, 0)),
            pl.BlockSpec((POOL_N, D_N), lambda i: (0, 0)),
            pl.BlockSpec((B_N, D_N), lambda i: (0, 0)),
            pl.BlockSpec((TH, D_N), lambda i: (enc_t(i), 0)),
            pl.BlockSpec((TH,), lambda i: (enc_t(i),)),
            pl.BlockSpec((D_N, TH), lambda i: (0, enc_t(i))),
            pl.BlockSpec((D_N,), lambda i: (0,)),
            pl.BlockSpec((D_N, TH), lambda i: (0, enc_t(i))),
            pl.BlockSpec((D_N,), lambda i: (0,)),
            pl.BlockSpec((TH, D_N), lambda i: (dec_t(i), 0)),
            pl.BlockSpec((TH,), lambda i: (dec_t(i),)),
            pl.BlockSpec((D_N, TH), lambda i: (0, dec_t(i))),
            pl.BlockSpec((D_N,), lambda i: (0,)),
        ],
        out_specs=[
            pl.BlockSpec((B_N, POOL_N), lambda i: (0, 0)),
            pl.BlockSpec((B_N, D_N), lambda i: (0, 0)),
            pl.BlockSpec((B_N, D_N), lambda i: (0, 0)),
            pl.BlockSpec((B_N, D_N), lambda i: (0, 0)),
            pl.BlockSpec((B_N, TOPK_N), lambda i: (0, 0)),
            pl.BlockSpec((1, 1), lambda i: (0, 0)),
        ],
        out_shape=[
            jax.ShapeDtypeStruct((B_N, POOL_N), jnp.float32),
            jax.ShapeDtypeStruct((B_N, D_N), jnp.float32),
            jax.ShapeDtypeStruct((B_N, D_N), jnp.float32),
            jax.ShapeDtypeStruct((B_N, D_N), jnp.float32),
            jax.ShapeDtypeStruct((B_N, TOPK_N), jnp.int32),
            jax.ShapeDtypeStruct((1, 1), jnp.float32),
        ],
        scratch_shapes=[
            pltpu.VMEM((B_N, D_N), jnp.float32),
            pltpu.VMEM((B_N, D_N), jnp.float32),
            pltpu.VMEM((B_N, D_N), jnp.float32),
            pltpu.VMEM((B_N, D_N), jnp.float32),
        ],
    )(cls_features, prompt_key, epsilon,
      W1, b1, Wm, bm, Wv, bv, D1, db1, D2, db2)

    prompted = pl.pallas_call(
        _assemble_body,
        grid_spec=pltpu.PrefetchScalarGridSpec(
            num_scalar_prefetch=1,
            grid=(NG,),
            in_specs=[
                pl.BlockSpec((POOL_N, LEN_N, D_N), lambda g, idx: (0, 0, 0)),
                pl.BlockSpec((BB, S_N, D_N), lambda g, idx: (g, 0, 0)),
                pl.BlockSpec((BB, 1, D_N), lambda g, idx: (g, 0, 0)),
                pl.BlockSpec((BB, 1, D_N), lambda g, idx: (g, 0, 0)),
            ],
            out_specs=pl.BlockSpec(memory_space=pl.ANY),
            scratch_shapes=[
                pltpu.VMEM((NBUF, BB, T_OUT, D_N), jnp.float32),
                pltpu.SemaphoreType.DMA((NBUF,)),
            ],
        ),
        out_shape=jax.ShapeDtypeStruct((B_N, T_OUT, D_N), jnp.float32),
    )(idx, prompt, x_embed,
      synth.reshape(B_N, 1, D_N), cls_features.reshape(B_N, 1, D_N))

    return (prompted, rs.reshape(()), sim, synth, mean, log_var, idx)


# speedup vs baseline: 1.6143x; 1.0111x over previous
import functools

import jax
import jax.numpy as jnp
from jax import lax
from jax.experimental import pallas as pl
from jax.experimental.pallas import tpu as pltpu

POOL_N = 64
TOPK_N = 8
LEN_N = 5
D_N = 768
H_N = 4096
B_N = 128
S_N = 197
T_OUT = 1 + TOPK_N * LEN_N + S_N
HEAD_N = TOPK_N * LEN_N

NT = 8
TH = H_N // NT

BB = 8
NG = B_N // BB
NBUF = 2


def _route_body(cls_ref, pk_ref, sim_ref, idx_ref, rs_ref):
    x = cls_ref[...]
    xn = x * lax.rsqrt(jnp.maximum(
        jnp.sum(x * x, axis=1, keepdims=True), 1e-12))
    p = pk_ref[...]
    pn = p * lax.rsqrt(jnp.maximum(
        jnp.sum(p * p, axis=1, keepdims=True), 1e-12))
    sim = lax.dot_general(xn, pn, (((1,), (1,)), ((), ())),
                          preferred_element_type=jnp.float32)
    sim_ref[...] = sim
    iot = lax.broadcasted_iota(jnp.int32, (B_N, POOL_N), 1)
    work = sim
    total = jnp.float32(0.0)
    cols = []
    for _ in range(TOPK_N):
        m = jnp.max(work, axis=1, keepdims=True)
        total = total + jnp.sum(m)
        cand = jnp.where(work == m, iot, POOL_N)
        aidx = jnp.min(cand, axis=1)
        cols.append(aidx.reshape(B_N, 1))
        work = jnp.where(iot == aidx[:, None], -jnp.inf, work)
    idx_ref[...] = jnp.concatenate(cols, axis=1)
    rs_ref[...] = jnp.reshape(total / jnp.float32(B_N), (1, 1))


def _fused_body(idx_sref,
                cls_ref, eps_ref, x_ref, clsrow_ref, prompt_ref,
                w1_ref, b1_ref, wm_ref, bm_ref, wv_ref, bv_ref,
                d1_ref, db1_ref, d2_ref, db2_ref,
                mean_ref, lv_ref, syn_ref, out_ref,
                acc_mean, acc_lv, z_ref, acc_syn, syn3, obuf, head, sem,
                hsem):
    g = pl.program_id(0)
    buf = lax.rem(g, NBUF)

    @pl.when(g == 0)
    def _init():
        acc_mean[...] = jnp.zeros_like(acc_mean)
        acc_lv[...] = jnp.zeros_like(acc_lv)
        acc_syn[...] = jnp.zeros_like(acc_syn)

    @pl.when(g < NT)
    def _encoder():
        h = jnp.maximum(
            lax.dot_general(cls_ref[...], w1_ref[...],
                            (((1,), (1,)), ((), ())),
                            preferred_element_type=jnp.float32)
            + b1_ref[...][None, :], 0.0)
        acc_mean[...] += lax.dot_general(h, wm_ref[...],
                                         (((1,), (1,)), ((), ())),
                                         preferred_element_type=jnp.float32)
        acc_lv[...] += lax.dot_general(h, wv_ref[...],
                                       (((1,), (1,)), ((), ())),
                                       preferred_element_type=jnp.float32)

    @pl.when(g == NT - 1)
    def _reparam():
        mean = acc_mean[...] + bm_ref[...][None, :]
        log_var = acc_lv[...] + bv_ref[...][None, :]
        mean_ref[...] = mean
        lv_ref[...] = log_var
        z_ref[...] = mean + jnp.exp(0.5 * log_var) * eps_ref[...]

    @pl.when(g >= NT)
    def _decoder():
        hd = jnp.maximum(
            lax.dot_general(z_ref[...], d1_ref[...],
                            (((1,), (1,)), ((), ())),
                            preferred_element_type=jnp.float32)
            + db1_ref[...][None, :], 0.0)
        acc_syn[...] += lax.dot_general(hd, d2_ref[...],
                                        (((1,), (1,)), ((), ())),
                                        preferred_element_type=jnp.float32)

    @pl.when(g >= NBUF)
    def _drain():
        pltpu.make_async_copy(
            obuf.at[buf],
            out_ref.at[pl.ds((g - NBUF) * BB, BB), pl.ds(HEAD_N, 198), :],
            sem.at[buf]).wait()

    obuf[buf, :, 1:, :] = x_ref[...]
    obuf[buf, :, 0:1, :] = clsrow_ref[...]
    pltpu.make_async_copy(
        obuf.at[buf],
        out_ref.at[pl.ds(g * BB, BB), pl.ds(HEAD_N, 198), :],
        sem.at[buf]).start()

    @pl.when(g == NG - 1)
    def _head():
        syn = acc_syn[...] + db2_ref[...][None, :]
        syn_ref[...] = syn
        syn3[...] = syn.reshape(B_N, 1, D_N)

        def bloop(b, carry):
            s = syn3[b]
            for k in range(TOPK_N):
                pidx = idx_sref[b, k]
                head[b, k * LEN_N:(k + 1) * LEN_N, :] = prompt_ref[pidx] + s
            return carry

        lax.fori_loop(0, B_N, bloop, 0)
        hcp = pltpu.make_async_copy(
            head, out_ref.at[:, pl.ds(0, HEAD_N), :], hsem)
        hcp.start()
        hcp.wait()
        for d in range(NBUF - 1, -1, -1):
            gg = NG - 1 - d
            pltpu.make_async_copy(
                obuf.at[lax.rem(jnp.int32(gg), NBUF)],
                out_ref.at[pl.ds(gg * BB, BB), pl.ds(HEAD_N, 198), :],
                sem.at[lax.rem(jnp.int32(gg), NBUF)]).wait()


def kernel(is_training, x_embed, cls_features, prompt, prompt_key, frequency,
           W1, b1, Wm, bm, Wv, bv, D1, db1, D2, db2, epsilon):
    del is_training, frequency

    sim, idx, rs = pl.pallas_call(
        _route_body,
        in_specs=[
            pl.BlockSpec((B_N, D_N), lambda: (0, 0)),
            pl.BlockSpec((POOL_N, D_N), lambda: (0, 0)),
        ],
        out_specs=[
            pl.BlockSpec((B_N, POOL_N), lambda: (0, 0)),
            pl.BlockSpec((B_N, TOPK_N), lambda: (0, 0)),
            pl.BlockSpec((1, 1), lambda: (0, 0)),
        ],
        out_shape=[
            jax.ShapeDtypeStruct((B_N, POOL_N), jnp.float32),
            jax.ShapeDtypeStruct((B_N, TOPK_N), jnp.int32),
            jax.ShapeDtypeStruct((1, 1), jnp.float32),
        ],
    )(cls_features, prompt_key)

    enc_t = lambda g, idxr: (jnp.where(g < NT, g, 0), 0)
    enc_t1 = lambda g, idxr: (jnp.where(g < NT, g, 0),)
    enc_tc = lambda g, idxr: (0, jnp.where(g < NT, g, 0))
    dec_t = lambda g, idxr: (jnp.where(g >= NT, g - NT, 0), 0)
    dec_t1 = lambda g, idxr: (jnp.where(g >= NT, g - NT, 0),)
    dec_tc = lambda g, idxr: (0, jnp.where(g >= NT, g - NT, 0))
    const2 = lambda g, idxr: (0, 0)
    const1 = lambda g, idxr: (0,)

    mean, log_var, synth, prompted = pl.pallas_call(
        _fused_body,
        grid_spec=pltpu.PrefetchScalarGridSpec(
            num_scalar_prefetch=1,
            grid=(NG,),
            in_specs=[
                pl.BlockSpec((B_N, D_N), const2),
                pl.BlockSpec((B_N, D_N), const2),
                pl.BlockSpec((BB, S_N, D_N), lambda g, i: (g, 0, 0)),
                pl.BlockSpec((BB, 1, D_N), lambda g, i: (g, 0, 0)),
                pl.BlockSpec((POOL_N, LEN_N, D_N),
                             lambda g, i: (0, 0, 0)),
                pl.BlockSpec((TH, D_N), enc_t),
                pl.BlockSpec((TH,), enc_t1),
                pl.BlockSpec((D_N, TH), enc_tc),
                pl.BlockSpec((D_N,), const1),
                pl.BlockSpec((D_N, TH), enc_tc),
                pl.BlockSpec((D_N,), const1),
                pl.BlockSpec((TH, D_N), dec_t),
                pl.BlockSpec((TH,), dec_t1),
                pl.BlockSpec((D_N, TH), dec_tc),
                pl.BlockSpec((D_N,), const1),
            ],
            out_specs=[
                pl.BlockSpec((B_N, D_N), const2),
                pl.BlockSpec((B_N, D_N), const2),
                pl.BlockSpec((B_N, D_N), const2),
                pl.BlockSpec(memory_space=pl.ANY),
            ],
            scratch_shapes=[
                pltpu.VMEM((B_N, D_N), jnp.float32),
                pltpu.VMEM((B_N, D_N), jnp.float32),
                pltpu.VMEM((B_N, D_N), jnp.float32),
                pltpu.VMEM((B_N, D_N), jnp.float32),
                pltpu.VMEM((B_N, 1, D_N), jnp.float32),
                pltpu.VMEM((NBUF, BB, 198, D_N), jnp.float32),
                pltpu.VMEM((B_N, HEAD_N, D_N), jnp.float32),
                pltpu.SemaphoreType.DMA((NBUF,)),
                pltpu.SemaphoreType.DMA,
            ],
        ),
        out_shape=[
            jax.ShapeDtypeStruct((B_N, D_N), jnp.float32),
            jax.ShapeDtypeStruct((B_N, D_N), jnp.float32),
            jax.ShapeDtypeStruct((B_N, D_N), jnp.float32),
            jax.ShapeDtypeStruct((B_N, T_OUT, D_N), jnp.float32),
        ],
    )(idx, cls_features, epsilon, x_embed,
      cls_features.reshape(B_N, 1, D_N), prompt,
      W1, b1, Wm, bm, Wv, bv, D1, db1, D2, db2)

    return (prompted, rs.reshape(()), sim, synth, mean, log_var, idx)


# speedup vs baseline: 1.6158x; 1.0009x over previous
import jax
import jax.numpy as jnp
from jax import lax
from jax.experimental import pallas as pl
from jax.experimental.pallas import tpu as pltpu

POOL_N = 64
TOPK_N = 8
LEN_N = 5
D_N = 768
H_N = 4096
B_N = 128
S_N = 197
T_OUT = 1 + TOPK_N * LEN_N + S_N
HEAD_N = TOPK_N * LEN_N

NT = 8
TH = H_N // NT

BB = 8
NG = B_N // BB
NBUF = 2


def _route_body(cls_ref, pk_ref, sim_ref, idx_ref, rs_ref):
    x = cls_ref[...]
    xn = x * lax.rsqrt(jnp.maximum(
        jnp.sum(x * x, axis=1, keepdims=True), 1e-12))
    p = pk_ref[...]
    pn = p * lax.rsqrt(jnp.maximum(
        jnp.sum(p * p, axis=1, keepdims=True), 1e-12))
    sim = lax.dot_general(xn, pn, (((1,), (1,)), ((), ())),
                          preferred_element_type=jnp.float32)
    sim_ref[...] = sim
    iot = lax.broadcasted_iota(jnp.int32, (B_N, POOL_N), 1)
    work = sim
    total = jnp.float32(0.0)
    cols = []
    for _ in range(TOPK_N):
        m = jnp.max(work, axis=1, keepdims=True)
        total = total + jnp.sum(m)
        cand = jnp.where(work == m, iot, POOL_N)
        aidx = jnp.min(cand, axis=1)
        cols.append(aidx.reshape(B_N, 1))
        work = jnp.where(iot == aidx[:, None], -jnp.inf, work)
    idx_ref[...] = jnp.concatenate(cols, axis=1)
    rs_ref[...] = jnp.reshape(total / jnp.float32(B_N), (1, 1))


def _fused_body(idx_sref,
                cls_ref, eps_ref, x_ref, clsrow_ref, prompt_ref,
                w1_ref, b1_ref, wm_ref, bm_ref, wv_ref, bv_ref,
                d1_ref, db1_ref, d2_ref, db2_ref,
                mean_ref, lv_ref, syn_ref, out_ref,
                acc_mean, acc_lv, z_ref, acc_syn, syn3, obuf, head, sem,
                hsem):
    g = pl.program_id(0)
    buf = lax.rem(g, NBUF)

    @pl.when(g == 0)
    def _init():
        acc_mean[...] = jnp.zeros_like(acc_mean)
        acc_lv[...] = jnp.zeros_like(acc_lv)
        acc_syn[...] = jnp.zeros_like(acc_syn)

    @pl.when(g < NT)
    def _encoder():
        h = jnp.maximum(
            lax.dot_general(cls_ref[...], w1_ref[...],
                            (((1,), (1,)), ((), ())),
                            preferred_element_type=jnp.float32)
            + b1_ref[...][None, :], 0.0)
        acc_mean[...] += lax.dot_general(h, wm_ref[...],
                                         (((1,), (1,)), ((), ())),
                                         preferred_element_type=jnp.float32)
        acc_lv[...] += lax.dot_general(h, wv_ref[...],
                                       (((1,), (1,)), ((), ())),
                                       preferred_element_type=jnp.float32)

    @pl.when(g == NT - 1)
    def _reparam():
        mean = acc_mean[...] + bm_ref[...][None, :]
        log_var = acc_lv[...] + bv_ref[...][None, :]
        mean_ref[...] = mean
        lv_ref[...] = log_var
        z_ref[...] = mean + jnp.exp(0.5 * log_var) * eps_ref[...]

    @pl.when(g >= NT)
    def _decoder():
        hd = jnp.maximum(
            lax.dot_general(z_ref[...], d1_ref[...],
                            (((1,), (1,)), ((), ())),
                            preferred_element_type=jnp.float32)
            + db1_ref[...][None, :], 0.0)
        acc_syn[...] += lax.dot_general(hd, d2_ref[...],
                                        (((1,), (1,)), ((), ())),
                                        preferred_element_type=jnp.float32)

    @pl.when(g >= NBUF)
    def _drain():
        pltpu.make_async_copy(
            obuf.at[buf],
            out_ref.at[pl.ds((g - NBUF) * BB, BB), pl.ds(HEAD_N, 198), :],
            sem.at[buf]).wait()

    obuf[buf, :, 1:, :] = x_ref[...]
    obuf[buf, :, 0:1, :] = clsrow_ref[...]
    pltpu.make_async_copy(
        obuf.at[buf],
        out_ref.at[pl.ds(g * BB, BB), pl.ds(HEAD_N, 198), :],
        sem.at[buf]).start()

    @pl.when(g == NG - 1)
    def _head():
        syn = acc_syn[...] + db2_ref[...][None, :]
        syn_ref[...] = syn
        syn3[...] = syn.reshape(B_N, 1, D_N)

        def bloop(b, carry):
            s = syn3[b]
            for k in range(TOPK_N):
                pidx = idx_sref[b, k]
                head[b, k * LEN_N:(k + 1) * LEN_N, :] = prompt_ref[pidx] + s
            return carry

        lax.fori_loop(0, B_N, bloop, 0)
        hcp = pltpu.make_async_copy(
            head, out_ref.at[:, pl.ds(0, HEAD_N), :], hsem)
        hcp.start()
        hcp.wait()
        for d in range(NBUF - 1, -1, -1):
            gg = NG - 1 - d
            pltpu.make_async_copy(
                obuf.at[lax.rem(jnp.int32(gg), NBUF)],
                out_ref.at[pl.ds(gg * BB, BB), pl.ds(HEAD_N, 198), :],
                sem.at[lax.rem(jnp.int32(gg), NBUF)]).wait()


def kernel(is_training, x_embed, cls_features, prompt, prompt_key, frequency,
           W1, b1, Wm, bm, Wv, bv, D1, db1, D2, db2, epsilon):
    del is_training, frequency

    sim, idx, rs = pl.pallas_call(
        _route_body,
        in_specs=[
            pl.BlockSpec((B_N, D_N), lambda: (0, 0)),
            pl.BlockSpec((POOL_N, D_N), lambda: (0, 0)),
        ],
        out_specs=[
            pl.BlockSpec((B_N, POOL_N), lambda: (0, 0)),
            pl.BlockSpec((B_N, TOPK_N), lambda: (0, 0)),
            pl.BlockSpec((1, 1), lambda: (0, 0)),
        ],
        out_shape=[
            jax.ShapeDtypeStruct((B_N, POOL_N), jnp.float32),
            jax.ShapeDtypeStruct((B_N, TOPK_N), jnp.int32),
            jax.ShapeDtypeStruct((1, 1), jnp.float32),
        ],
    )(cls_features, prompt_key)

    enc_t = lambda g, idxr: (jnp.where(g < NT, g, 0), 0)
    enc_t1 = lambda g, idxr: (jnp.where(g < NT, g, 0),)
    enc_tc = lambda g, idxr: (0, jnp.where(g < NT, g, 0))
    dec_t = lambda g, idxr: (jnp.where(g >= NT, g - NT, 0), 0)
    dec_t1 = lambda g, idxr: (jnp.where(g >= NT, g - NT, 0),)
    dec_tc = lambda g, idxr: (0, jnp.where(g >= NT, g - NT, 0))
    const2 = lambda g, idxr: (0, 0)
    const1 = lambda g, idxr: (0,)

    mean, log_var, synth, prompted = pl.pallas_call(
        _fused_body,
        grid_spec=pltpu.PrefetchScalarGridSpec(
            num_scalar_prefetch=1,
            grid=(NG,),
            in_specs=[
                pl.BlockSpec((B_N, D_N), const2),
                pl.BlockSpec((B_N, D_N), const2),
                pl.BlockSpec((BB, S_N, D_N), lambda g, i: (g, 0, 0)),
                pl.BlockSpec((BB, 1, D_N), lambda g, i: (g, 0, 0)),
                pl.BlockSpec((POOL_N, LEN_N, D_N),
                             lambda g, i: (0, 0, 0)),
                pl.BlockSpec((TH, D_N), enc_t),
                pl.BlockSpec((TH,), enc_t1),
                pl.BlockSpec((D_N, TH), enc_tc),
                pl.BlockSpec((D_N,), const1),
                pl.BlockSpec((D_N, TH), enc_tc),
                pl.BlockSpec((D_N,), const1),
                pl.BlockSpec((TH, D_N), dec_t),
                pl.BlockSpec((TH,), dec_t1),
                pl.BlockSpec((D_N, TH), dec_tc),
                pl.BlockSpec((D_N,), const1),
            ],
            out_specs=[
                pl.BlockSpec((B_N, D_N), const2),
                pl.BlockSpec((B_N, D_N), const2),
                pl.BlockSpec((B_N, D_N), const2),
                pl.BlockSpec(memory_space=pl.ANY),
            ],
            scratch_shapes=[
                pltpu.VMEM((B_N, D_N), jnp.float32),
                pltpu.VMEM((B_N, D_N), jnp.float32),
                pltpu.VMEM((B_N, D_N), jnp.float32),
                pltpu.VMEM((B_N, D_N), jnp.float32),
                pltpu.VMEM((B_N, 1, D_N), jnp.float32),
                pltpu.VMEM((NBUF, BB, 198, D_N), jnp.float32),
                pltpu.VMEM((B_N, HEAD_N, D_N), jnp.float32),
                pltpu.SemaphoreType.DMA((NBUF,)),
                pltpu.SemaphoreType.DMA,
            ],
        ),
        out_shape=[
            jax.ShapeDtypeStruct((B_N, D_N), jnp.float32),
            jax.ShapeDtypeStruct((B_N, D_N), jnp.float32),
            jax.ShapeDtypeStruct((B_N, D_N), jnp.float32),
            jax.ShapeDtypeStruct((B_N, T_OUT, D_N), jnp.float32),
        ],
    )(idx, cls_features, epsilon, x_embed,
      cls_features.reshape(B_N, 1, D_N), prompt,
      W1, b1, Wm, bm, Wv, bv, D1, db1, D2, db2)

    return (prompted, rs.reshape(()), sim, synth, mean, log_var, idx)
